# Initial kernel scaffold; baseline (speedup 1.0000x reference)
import jax
import jax.numpy as jnp
from jax.experimental import pallas as pl


def kernel(x, pos, Z, batch, W0, b0, W1, b1):
    raise NotImplementedError("write your pallas kernel here")


# speedup vs baseline: 8.8846x; 8.8846x over previous
import functools

import jax
import jax.numpy as jnp
from jax import lax
from jax.experimental import pallas as pl
from jax.experimental.pallas import tpu as pltpu
from jax.experimental.pallas import tpu_sc as plsc

_B = 4096
_N = 320000
_D = 128
_H = 64

_MEAN = 0.7546106515883616
_STD = 0.30338715545464656
_A_TO_A0 = 1.8897268777743552
_A2 = _A_TO_A0 * _A_TO_A0

_MASS_BY_Z = ((1, 1.00784), (6, 12.0107), (7, 14.0067), (8, 15.999),
              (9, 18.998403))


def _mass_of(zv):
    m = jnp.zeros(zv.shape, jnp.float32)
    for z, mz in _MASS_BY_Z:
        m = jnp.where(zv == z, jnp.float32(mz), m)
    return m

_BLK = 2560
_NBLK = _N // _BLK


def _mlp_body(x_ref, w0_ref, b0_ref, w1_ref, b1_ref, o_ref):
    xb = x_ref[...]
    h = lax.dot_general(xb, w0_ref[...], (((1,), (1,)), ((), ())),
                        preferred_element_type=jnp.float32,
                        precision=lax.Precision.HIGHEST)
    h = h + b0_ref[...]
    s = h * (1.0 / (1.0 + jnp.exp(-h)))
    q = jnp.sum(s * w1_ref[...], axis=1, keepdims=True) + b1_ref[...]
    o_ref[...] = q * _STD + _MEAN


def _mlp(x, w0, b0, w1, b1):
    return pl.pallas_call(
        _mlp_body,
        grid=(_NBLK,),
        in_specs=[
            pl.BlockSpec((_BLK, _D), lambda i: (i, 0)),
            pl.BlockSpec((_H, _D), lambda i: (0, 0)),
            pl.BlockSpec((1, _H), lambda i: (0, 0)),
            pl.BlockSpec((1, _H), lambda i: (0, 0)),
            pl.BlockSpec((1, 1), lambda i: (0, 0)),
        ],
        out_specs=pl.BlockSpec((_BLK, 1), lambda i: (i, 0)),
        out_shape=jax.ShapeDtypeStruct((_N, 1), jnp.float32),
    )(x, w0, b0, w1, b1)


_NC = 2
_NS = 16
_L = 16
_AT = _N // _NS
_CH = 4000
_NCH = _AT // _CH
_NG = _CH // _L
_MSL = _B // _NS


def _seg_scatter_add(acc, b, b_next, endmask, midmask, v):
    s = plsc.cumsum(v)
    plsc.addupdate_scatter(acc, [b], s, mask=endmask)
    plsc.addupdate_scatter(acc, [b_next], -s, mask=midmask)


def _sc_body(batch_hbm, z_hbm, px_hbm, py_hbm, pz_hbm, q_hbm, out_hbm,
             bufb, bufz, bufpx, bufpy, bufpz, bufq,
             accm, accx, accy, accz, accq, accn,
             molloc, r2loc, redbuf, sums, tmpw, res,
             spacc, spmol, spr2):
    cid = lax.axis_index("c")
    sid = lax.axis_index("s")
    base = sid * _AT
    iot = lax.iota(jnp.int32, _L)
    shift_idx = jnp.minimum(iot + 1, _L - 1)
    lastlane = iot == (_L - 1)
    notlast = iot < (_L - 1)
    zero16 = jnp.zeros((_L,), jnp.float32)
    one16 = jnp.ones((_L,), jnp.float32)
    rows0 = jnp.full((_L,), 0, jnp.int32)
    rows1 = jnp.full((_L,), 1, jnp.int32)
    rows2 = jnp.full((_L,), 2, jnp.int32)
    rows3 = jnp.full((_L,), 3, jnp.int32)

    def _zero(i, _):
        o = i * _L
        accm[pl.ds(o, _L)] = zero16
        accx[pl.ds(o, _L)] = zero16
        accy[pl.ds(o, _L)] = zero16
        accz[pl.ds(o, _L)] = zero16
        accq[pl.ds(o, _L)] = zero16
        accn[pl.ds(o, _L)] = zero16
        r2loc[pl.ds(o, _L)] = zero16
        return 0
    lax.fori_loop(0, _B // _L, _zero, 0)

    def _chunk_a(k, _):
        a0 = base + k * _CH
        pltpu.sync_copy(batch_hbm.at[pl.ds(a0, _CH)], bufb)
        pltpu.sync_copy(z_hbm.at[pl.ds(a0, _CH)], bufz)
        pltpu.sync_copy(px_hbm.at[pl.ds(a0, _CH)], bufpx)
        pltpu.sync_copy(py_hbm.at[pl.ds(a0, _CH)], bufpy)
        pltpu.sync_copy(pz_hbm.at[pl.ds(a0, _CH)], bufpz)
        pltpu.sync_copy(q_hbm.at[pl.ds(a0, _CH)], bufq)

        def _grp(g, _):
            o = g * _L
            b = bufb[pl.ds(o, _L)]
            zv = bufz[pl.ds(o, _L)]
            m = _mass_of(zv)
            px = bufpx[pl.ds(o, _L)]
            py = bufpy[pl.ds(o, _L)]
            pz = bufpz[pl.ds(o, _L)]
            qv = bufq[pl.ds(o, _L)]
            b_next = b.at[shift_idx].get(mode="promise_in_bounds")
            diff = b != b_next
            endmask = diff | lastlane
            midmask = diff & notlast
            _seg_scatter_add(accm, b, b_next, endmask, midmask, m)
            _seg_scatter_add(accx, b, b_next, endmask, midmask, m * px)
            _seg_scatter_add(accy, b, b_next, endmask, midmask, m * py)
            _seg_scatter_add(accz, b, b_next, endmask, midmask, m * pz)
            _seg_scatter_add(accq, b, b_next, endmask, midmask, qv)
            _seg_scatter_add(accn, b, b_next, endmask, midmask, one16)
            return 0
        lax.fori_loop(0, _NG, _grp, 0)
        return 0
    lax.fori_loop(0, _NCH, _chunk_a, 0)

    for c, a in enumerate((accm, accx, accy, accz, accq, accn)):
        pltpu.sync_copy(a, spacc.at[c, sid])
    plsc.subcore_barrier()

    m0 = sid * _MSL
    for c in range(6):
        for t in range(_NS):
            pltpu.sync_copy(spacc.at[c, t, pl.ds(m0, _MSL)], redbuf.at[t])

        def _red(g, _, c=c):
            o = g * _L
            v = zero16
            for t in range(_NS):
                v = v + redbuf[t, pl.ds(o, _L)]
            sums[c, pl.ds(o, _L)] = v
            return 0
        lax.fori_loop(0, _MSL // _L, _red, 0)

    def _mol(g, _):
        o = g * _L
        ms = sums[0, pl.ds(o, _L)]
        tmpw[0, pl.ds(o, _L)] = sums[1, pl.ds(o, _L)] / ms
        tmpw[1, pl.ds(o, _L)] = sums[2, pl.ds(o, _L)] / ms
        tmpw[2, pl.ds(o, _L)] = sums[3, pl.ds(o, _L)] / ms
        tmpw[3, pl.ds(o, _L)] = sums[4, pl.ds(o, _L)] / sums[5, pl.ds(o, _L)]
        return 0
    lax.fori_loop(0, _MSL // _L, _mol, 0)
    for c in range(4):
        pltpu.sync_copy(tmpw.at[c], spmol.at[c, pl.ds(m0, _MSL)])
    plsc.subcore_barrier()
    pltpu.sync_copy(spmol, molloc)

    def _chunk_b(k, _):
        a0 = base + k * _CH
        pltpu.sync_copy(batch_hbm.at[pl.ds(a0, _CH)], bufb)
        pltpu.sync_copy(z_hbm.at[pl.ds(a0, _CH)], bufz)
        pltpu.sync_copy(px_hbm.at[pl.ds(a0, _CH)], bufpx)
        pltpu.sync_copy(py_hbm.at[pl.ds(a0, _CH)], bufpy)
        pltpu.sync_copy(pz_hbm.at[pl.ds(a0, _CH)], bufpz)
        pltpu.sync_copy(q_hbm.at[pl.ds(a0, _CH)], bufq)

        def _grp(g, _):
            o = g * _L
            b = bufb[pl.ds(o, _L)]
            zv = bufz[pl.ds(o, _L)]
            zf = zv.astype(jnp.float32)
            px = bufpx[pl.ds(o, _L)]
            py = bufpy[pl.ds(o, _L)]
            pz = bufpz[pl.ds(o, _L)]
            qv = bufq[pl.ds(o, _L)]
            cmx = plsc.load_gather(molloc, [rows0, b])
            cmy = plsc.load_gather(molloc, [rows1, b])
            cmz = plsc.load_gather(molloc, [rows2, b])
            mq = plsc.load_gather(molloc, [rows3, b])
            dx = px - cmx
            dy = py - cmy
            dz = pz - cmz
            r2v = (dx * dx + dy * dy + dz * dz) * _A2
            cloud = jnp.abs(qv - mq - zf)
            b_next = b.at[shift_idx].get(mode="promise_in_bounds")
            diff = b != b_next
            _seg_scatter_add(r2loc, b, b_next, diff | lastlane,
                             diff & notlast, cloud * r2v)
            return 0
        lax.fori_loop(0, _NG, _grp, 0)
        return 0
    lax.fori_loop(0, _NCH, _chunk_b, 0)

    pltpu.sync_copy(r2loc, spr2.at[sid])
    plsc.subcore_barrier()
    for t in range(_NS):
        pltpu.sync_copy(spr2.at[t, pl.ds(m0, _MSL)], redbuf.at[t])

    def _redr(g, _):
        o = g * _L
        v = zero16
        for t in range(_NS):
            v = v + redbuf[t, pl.ds(o, _L)]
        res[pl.ds(o, _L)] = v
        return 0
    lax.fori_loop(0, _MSL // _L, _redr, 0)

    @pl.when(sid // (_NS // _NC) == cid)
    def _():
        pltpu.sync_copy(res, out_hbm.at[pl.ds(m0, _MSL)])


_sc_call = pl.kernel(
    _sc_body,
    out_type=jax.ShapeDtypeStruct((_B,), jnp.float32),
    mesh=plsc.VectorSubcoreMesh(core_axis_name="c", subcore_axis_name="s",
                                num_cores=_NC, num_subcores=_NS),
    compiler_params=pltpu.CompilerParams(needs_layout_passes=False),
    scratch_types=[
        pltpu.VMEM((_CH,), jnp.int32),
        pltpu.VMEM((_CH,), jnp.int32),
        pltpu.VMEM((_CH,), jnp.float32),
        pltpu.VMEM((_CH,), jnp.float32),
        pltpu.VMEM((_CH,), jnp.float32),
        pltpu.VMEM((_CH,), jnp.float32),
        pltpu.VMEM((_B,), jnp.float32),
        pltpu.VMEM((_B,), jnp.float32),
        pltpu.VMEM((_B,), jnp.float32),
        pltpu.VMEM((_B,), jnp.float32),
        pltpu.VMEM((_B,), jnp.float32),
        pltpu.VMEM((_B,), jnp.float32),
        pltpu.VMEM((4, _B), jnp.float32),
        pltpu.VMEM((_B,), jnp.float32),
        pltpu.VMEM((_NS, _MSL), jnp.float32),
        pltpu.VMEM((6, _MSL), jnp.float32),
        pltpu.VMEM((4, _MSL), jnp.float32),
        pltpu.VMEM((_MSL,), jnp.float32),
        pltpu.VMEM_SHARED((6, _NS, _B), jnp.float32),
        pltpu.VMEM_SHARED((4, _B), jnp.float32),
        pltpu.VMEM_SHARED((_NS, _B), jnp.float32),
    ],
)


def kernel(x, pos, Z, batch, W0, b0, W1, b1):
    q = _mlp(x, W0, b0.reshape(1, _H), W1, b1.reshape(1, 1))
    pos_t = pos.T
    r2 = _sc_call(batch.astype(jnp.int32), Z[:, 0].astype(jnp.int32),
                  pos_t[0], pos_t[1], pos_t[2], q[:, 0])
    return r2.reshape(_B, 1)


# speedup vs baseline: 8.8849x; 1.0000x over previous
import functools

import jax
import jax.numpy as jnp
from jax import lax
from jax.experimental import pallas as pl
from jax.experimental.pallas import tpu as pltpu
from jax.experimental.pallas import tpu_sc as plsc

_B = 4096
_N = 320000
_D = 128
_H = 64

_MEAN = 0.7546106515883616
_STD = 0.30338715545464656
_A_TO_A0 = 1.8897268777743552
_A2 = _A_TO_A0 * _A_TO_A0

_MASS_BY_Z = ((1, 1.00784), (6, 12.0107), (7, 14.0067), (8, 15.999),
              (9, 18.998403))


def _mass_of(zv):
    m = jnp.zeros(zv.shape, jnp.float32)
    for z, mz in _MASS_BY_Z:
        m = jnp.where(zv == z, jnp.float32(mz), m)
    return m

_BLK = 2560
_NBLK = _N // _BLK


def _mlp_body(x_ref, w0_ref, b0_ref, w1_ref, b1_ref, o_ref):
    xb = x_ref[...]
    h = lax.dot_general(xb, w0_ref[...], (((1,), (1,)), ((), ())),
                        preferred_element_type=jnp.float32,
                        precision=lax.Precision.HIGHEST)
    h = h + b0_ref[...]
    s = h * (1.0 / (1.0 + jnp.exp(-h)))
    q = jnp.sum(s * w1_ref[...], axis=1, keepdims=True) + b1_ref[...]
    o_ref[...] = q * _STD + _MEAN


def _mlp(x, w0, b0, w1, b1):
    return pl.pallas_call(
        _mlp_body,
        grid=(_NBLK,),
        in_specs=[
            pl.BlockSpec((_BLK, _D), lambda i: (i, 0)),
            pl.BlockSpec((_H, _D), lambda i: (0, 0)),
            pl.BlockSpec((1, _H), lambda i: (0, 0)),
            pl.BlockSpec((1, _H), lambda i: (0, 0)),
            pl.BlockSpec((1, 1), lambda i: (0, 0)),
        ],
        out_specs=pl.BlockSpec((_BLK, 1), lambda i: (i, 0)),
        out_shape=jax.ShapeDtypeStruct((_N, 1), jnp.float32),
        compiler_params=pltpu.CompilerParams(
            dimension_semantics=("parallel",)),
    )(x, w0, b0, w1, b1)


_NC = 2
_NS = 16
_L = 16
_AT = _N // _NS
_CH = 4000
_NCH = _AT // _CH
_NG = _CH // _L
_MSL = _B // _NS


def _seg_scatter_add(acc, b, b_next, endmask, midmask, v):
    s = plsc.cumsum(v)
    plsc.addupdate_scatter(acc, [b], s, mask=endmask)
    plsc.addupdate_scatter(acc, [b_next], -s, mask=midmask)


def _sc_body(batch_hbm, z_hbm, px_hbm, py_hbm, pz_hbm, q_hbm, out_hbm,
             bufb, bufz, bufpx, bufpy, bufpz, bufq,
             accm, accx, accy, accz, accq, accn,
             molloc, r2loc, redbuf, sums, tmpw, res,
             spacc, spmol, spr2):
    cid = lax.axis_index("c")
    sid = lax.axis_index("s")
    base = sid * _AT
    iot = lax.iota(jnp.int32, _L)
    shift_idx = jnp.minimum(iot + 1, _L - 1)
    lastlane = iot == (_L - 1)
    notlast = iot < (_L - 1)
    zero16 = jnp.zeros((_L,), jnp.float32)
    one16 = jnp.ones((_L,), jnp.float32)
    rows0 = jnp.full((_L,), 0, jnp.int32)
    rows1 = jnp.full((_L,), 1, jnp.int32)
    rows2 = jnp.full((_L,), 2, jnp.int32)
    rows3 = jnp.full((_L,), 3, jnp.int32)

    def _zero(i, _):
        o = i * _L
        accm[pl.ds(o, _L)] = zero16
        accx[pl.ds(o, _L)] = zero16
        accy[pl.ds(o, _L)] = zero16
        accz[pl.ds(o, _L)] = zero16
        accq[pl.ds(o, _L)] = zero16
        accn[pl.ds(o, _L)] = zero16
        r2loc[pl.ds(o, _L)] = zero16
        return 0
    lax.fori_loop(0, _B // _L, _zero, 0)

    def _chunk_a(k, _):
        a0 = base + k * _CH
        pltpu.sync_copy(batch_hbm.at[pl.ds(a0, _CH)], bufb)
        pltpu.sync_copy(z_hbm.at[pl.ds(a0, _CH)], bufz)
        pltpu.sync_copy(px_hbm.at[pl.ds(a0, _CH)], bufpx)
        pltpu.sync_copy(py_hbm.at[pl.ds(a0, _CH)], bufpy)
        pltpu.sync_copy(pz_hbm.at[pl.ds(a0, _CH)], bufpz)
        pltpu.sync_copy(q_hbm.at[pl.ds(a0, _CH)], bufq)

        def _grp(g, _):
            o = g * _L
            b = bufb[pl.ds(o, _L)]
            zv = bufz[pl.ds(o, _L)]
            m = _mass_of(zv)
            px = bufpx[pl.ds(o, _L)]
            py = bufpy[pl.ds(o, _L)]
            pz = bufpz[pl.ds(o, _L)]
            qv = bufq[pl.ds(o, _L)]
            b_next = b.at[shift_idx].get(mode="promise_in_bounds")
            diff = b != b_next
            endmask = diff | lastlane
            midmask = diff & notlast
            _seg_scatter_add(accm, b, b_next, endmask, midmask, m)
            _seg_scatter_add(accx, b, b_next, endmask, midmask, m * px)
            _seg_scatter_add(accy, b, b_next, endmask, midmask, m * py)
            _seg_scatter_add(accz, b, b_next, endmask, midmask, m * pz)
            _seg_scatter_add(accq, b, b_next, endmask, midmask, qv)
            _seg_scatter_add(accn, b, b_next, endmask, midmask, one16)
            return 0
        lax.fori_loop(0, _NG, _grp, 0)
        return 0
    lax.fori_loop(0, _NCH, _chunk_a, 0)

    for c, a in enumerate((accm, accx, accy, accz, accq, accn)):
        pltpu.sync_copy(a, spacc.at[c, sid])
    plsc.subcore_barrier()

    m0 = sid * _MSL
    for c in range(6):
        for t in range(_NS):
            pltpu.sync_copy(spacc.at[c, t, pl.ds(m0, _MSL)], redbuf.at[t])

        def _red(g, _, c=c):
            o = g * _L
            v = zero16
            for t in range(_NS):
                v = v + redbuf[t, pl.ds(o, _L)]
            sums[c, pl.ds(o, _L)] = v
            return 0
        lax.fori_loop(0, _MSL // _L, _red, 0)

    def _mol(g, _):
        o = g * _L
        ms = sums[0, pl.ds(o, _L)]
        tmpw[0, pl.ds(o, _L)] = sums[1, pl.ds(o, _L)] / ms
        tmpw[1, pl.ds(o, _L)] = sums[2, pl.ds(o, _L)] / ms
        tmpw[2, pl.ds(o, _L)] = sums[3, pl.ds(o, _L)] / ms
        tmpw[3, pl.ds(o, _L)] = sums[4, pl.ds(o, _L)] / sums[5, pl.ds(o, _L)]
        return 0
    lax.fori_loop(0, _MSL // _L, _mol, 0)
    for c in range(4):
        pltpu.sync_copy(tmpw.at[c], spmol.at[c, pl.ds(m0, _MSL)])
    plsc.subcore_barrier()
    pltpu.sync_copy(spmol, molloc)

    def _chunk_b(k, _):
        a0 = base + k * _CH
        pltpu.sync_copy(batch_hbm.at[pl.ds(a0, _CH)], bufb)
        pltpu.sync_copy(z_hbm.at[pl.ds(a0, _CH)], bufz)
        pltpu.sync_copy(px_hbm.at[pl.ds(a0, _CH)], bufpx)
        pltpu.sync_copy(py_hbm.at[pl.ds(a0, _CH)], bufpy)
        pltpu.sync_copy(pz_hbm.at[pl.ds(a0, _CH)], bufpz)
        pltpu.sync_copy(q_hbm.at[pl.ds(a0, _CH)], bufq)

        def _grp(g, _):
            o = g * _L
            b = bufb[pl.ds(o, _L)]
            zv = bufz[pl.ds(o, _L)]
            zf = zv.astype(jnp.float32)
            px = bufpx[pl.ds(o, _L)]
            py = bufpy[pl.ds(o, _L)]
            pz = bufpz[pl.ds(o, _L)]
            qv = bufq[pl.ds(o, _L)]
            cmx = plsc.load_gather(molloc, [rows0, b])
            cmy = plsc.load_gather(molloc, [rows1, b])
            cmz = plsc.load_gather(molloc, [rows2, b])
            mq = plsc.load_gather(molloc, [rows3, b])
            dx = px - cmx
            dy = py - cmy
            dz = pz - cmz
            r2v = (dx * dx + dy * dy + dz * dz) * _A2
            cloud = jnp.abs(qv - mq - zf)
            b_next = b.at[shift_idx].get(mode="promise_in_bounds")
            diff = b != b_next
            _seg_scatter_add(r2loc, b, b_next, diff | lastlane,
                             diff & notlast, cloud * r2v)
            return 0
        lax.fori_loop(0, _NG, _grp, 0)
        return 0
    lax.fori_loop(0, _NCH, _chunk_b, 0)

    pltpu.sync_copy(r2loc, spr2.at[sid])
    plsc.subcore_barrier()
    for t in range(_NS):
        pltpu.sync_copy(spr2.at[t, pl.ds(m0, _MSL)], redbuf.at[t])

    def _redr(g, _):
        o = g * _L
        v = zero16
        for t in range(_NS):
            v = v + redbuf[t, pl.ds(o, _L)]
        res[pl.ds(o, _L)] = v
        return 0
    lax.fori_loop(0, _MSL // _L, _redr, 0)

    @pl.when(sid // (_NS // _NC) == cid)
    def _():
        pltpu.sync_copy(res, out_hbm.at[pl.ds(m0, _MSL)])


_sc_call = pl.kernel(
    _sc_body,
    out_type=jax.ShapeDtypeStruct((_B,), jnp.float32),
    mesh=plsc.VectorSubcoreMesh(core_axis_name="c", subcore_axis_name="s",
                                num_cores=_NC, num_subcores=_NS),
    compiler_params=pltpu.CompilerParams(needs_layout_passes=False),
    scratch_types=[
        pltpu.VMEM((_CH,), jnp.int32),
        pltpu.VMEM((_CH,), jnp.int32),
        pltpu.VMEM((_CH,), jnp.float32),
        pltpu.VMEM((_CH,), jnp.float32),
        pltpu.VMEM((_CH,), jnp.float32),
        pltpu.VMEM((_CH,), jnp.float32),
        pltpu.VMEM((_B,), jnp.float32),
        pltpu.VMEM((_B,), jnp.float32),
        pltpu.VMEM((_B,), jnp.float32),
        pltpu.VMEM((_B,), jnp.float32),
        pltpu.VMEM((_B,), jnp.float32),
        pltpu.VMEM((_B,), jnp.float32),
        pltpu.VMEM((4, _B), jnp.float32),
        pltpu.VMEM((_B,), jnp.float32),
        pltpu.VMEM((_NS, _MSL), jnp.float32),
        pltpu.VMEM((6, _MSL), jnp.float32),
        pltpu.VMEM((4, _MSL), jnp.float32),
        pltpu.VMEM((_MSL,), jnp.float32),
        pltpu.VMEM_SHARED((6, _NS, _B), jnp.float32),
        pltpu.VMEM_SHARED((4, _B), jnp.float32),
        pltpu.VMEM_SHARED((_NS, _B), jnp.float32),
    ],
)


def kernel(x, pos, Z, batch, W0, b0, W1, b1):
    q = _mlp(x, W0, b0.reshape(1, _H), W1, b1.reshape(1, 1))
    pos_t = pos.T
    r2 = _sc_call(batch.astype(jnp.int32), Z[:, 0].astype(jnp.int32),
                  pos_t[0], pos_t[1], pos_t[2], q[:, 0])
    return r2.reshape(_B, 1)


# speedup vs baseline: 9.6180x; 1.0825x over previous
import functools

import jax
import jax.numpy as jnp
from jax import lax
from jax.experimental import pallas as pl
from jax.experimental.pallas import tpu as pltpu
from jax.experimental.pallas import tpu_sc as plsc

_B = 4096
_N = 320000
_D = 128
_H = 64

_MEAN = 0.7546106515883616
_STD = 0.30338715545464656
_A_TO_A0 = 1.8897268777743552
_A2 = _A_TO_A0 * _A_TO_A0

_MASS_BY_Z = ((1, 1.00784), (6, 12.0107), (7, 14.0067), (8, 15.999),
              (9, 18.998403))


def _mass_of(zv):
    m = jnp.zeros(zv.shape, jnp.float32)
    for z, mz in _MASS_BY_Z:
        m = jnp.where(zv == z, jnp.float32(mz), m)
    return m

_BLK = 8000
_NBLK = _N // _BLK


def _mlp_body(x_ref, w0_ref, b0_ref, w1_ref, b1_ref, o_ref):
    xb = x_ref[...]
    h = lax.dot_general(xb, w0_ref[...], (((1,), (1,)), ((), ())),
                        preferred_element_type=jnp.float32,
                        precision=lax.Precision.HIGHEST)
    h = h + b0_ref[...]
    s = h * (1.0 / (1.0 + jnp.exp(-h)))
    q = jnp.sum(s * w1_ref[...], axis=1, keepdims=True) + b1_ref[...]
    o_ref[...] = q * _STD + _MEAN


def _mlp(x, w0, b0, w1, b1):
    return pl.pallas_call(
        _mlp_body,
        grid=(_NBLK,),
        in_specs=[
            pl.BlockSpec((_BLK, _D), lambda i: (i, 0)),
            pl.BlockSpec((_H, _D), lambda i: (0, 0)),
            pl.BlockSpec((1, _H), lambda i: (0, 0)),
            pl.BlockSpec((1, _H), lambda i: (0, 0)),
            pl.BlockSpec((1, 1), lambda i: (0, 0)),
        ],
        out_specs=pl.BlockSpec((_BLK, 1), lambda i: (i, 0)),
        out_shape=jax.ShapeDtypeStruct((_N, 1), jnp.float32),
        compiler_params=pltpu.CompilerParams(
            dimension_semantics=("parallel",)),
    )(x, w0, b0, w1, b1)


_NC = 2
_NS = 16
_L = 16
_AT = _N // _NS
_CH = 4000
_NCH = _AT // _CH
_NG = _CH // _L
_MSL = _B // _NS


def _seg_scatter_add(acc, b, b_next, endmask, midmask, v):
    s = plsc.cumsum(v)
    plsc.addupdate_scatter(acc, [b], s, mask=endmask)
    plsc.addupdate_scatter(acc, [b_next], -s, mask=midmask)


def _sc_body(batch_hbm, z_hbm, px_hbm, py_hbm, pz_hbm, q_hbm, out_hbm,
             bufb, bufz, bufpx, bufpy, bufpz, bufq,
             accm, accx, accy, accz, accq, accn,
             molloc, r2loc, redbuf, sums, tmpw, res,
             spacc, spmol, spr2):
    cid = lax.axis_index("c")
    sid = lax.axis_index("s")
    base = sid * _AT
    iot = lax.iota(jnp.int32, _L)
    shift_idx = jnp.minimum(iot + 1, _L - 1)
    lastlane = iot == (_L - 1)
    notlast = iot < (_L - 1)
    zero16 = jnp.zeros((_L,), jnp.float32)
    one16 = jnp.ones((_L,), jnp.float32)
    rows0 = jnp.full((_L,), 0, jnp.int32)
    rows1 = jnp.full((_L,), 1, jnp.int32)
    rows2 = jnp.full((_L,), 2, jnp.int32)
    rows3 = jnp.full((_L,), 3, jnp.int32)

    def _zero(i, _):
        o = i * _L
        accm[pl.ds(o, _L)] = zero16
        accx[pl.ds(o, _L)] = zero16
        accy[pl.ds(o, _L)] = zero16
        accz[pl.ds(o, _L)] = zero16
        accq[pl.ds(o, _L)] = zero16
        accn[pl.ds(o, _L)] = zero16
        r2loc[pl.ds(o, _L)] = zero16
        return 0
    lax.fori_loop(0, _B // _L, _zero, 0)

    def _chunk_a(k, _):
        a0 = base + k * _CH
        pltpu.sync_copy(batch_hbm.at[pl.ds(a0, _CH)], bufb)
        pltpu.sync_copy(z_hbm.at[pl.ds(a0, _CH)], bufz)
        pltpu.sync_copy(px_hbm.at[pl.ds(a0, _CH)], bufpx)
        pltpu.sync_copy(py_hbm.at[pl.ds(a0, _CH)], bufpy)
        pltpu.sync_copy(pz_hbm.at[pl.ds(a0, _CH)], bufpz)
        pltpu.sync_copy(q_hbm.at[pl.ds(a0, _CH)], bufq)

        def _grp(g, _):
            o = g * _L
            b = bufb[pl.ds(o, _L)]
            zv = bufz[pl.ds(o, _L)]
            m = _mass_of(zv)
            px = bufpx[pl.ds(o, _L)]
            py = bufpy[pl.ds(o, _L)]
            pz = bufpz[pl.ds(o, _L)]
            qv = bufq[pl.ds(o, _L)]
            b_next = b.at[shift_idx].get(mode="promise_in_bounds")
            diff = b != b_next
            endmask = diff | lastlane
            midmask = diff & notlast
            _seg_scatter_add(accm, b, b_next, endmask, midmask, m)
            _seg_scatter_add(accx, b, b_next, endmask, midmask, m * px)
            _seg_scatter_add(accy, b, b_next, endmask, midmask, m * py)
            _seg_scatter_add(accz, b, b_next, endmask, midmask, m * pz)
            _seg_scatter_add(accq, b, b_next, endmask, midmask, qv)
            _seg_scatter_add(accn, b, b_next, endmask, midmask, one16)
            return 0
        lax.fori_loop(0, _NG, _grp, 0)
        return 0
    lax.fori_loop(0, _NCH, _chunk_a, 0)

    for c, a in enumerate((accm, accx, accy, accz, accq, accn)):
        pltpu.sync_copy(a, spacc.at[c, sid])
    plsc.subcore_barrier()

    m0 = sid * _MSL
    for c in range(6):
        for t in range(_NS):
            pltpu.sync_copy(spacc.at[c, t, pl.ds(m0, _MSL)], redbuf.at[t])

        def _red(g, _, c=c):
            o = g * _L
            v = zero16
            for t in range(_NS):
                v = v + redbuf[t, pl.ds(o, _L)]
            sums[c, pl.ds(o, _L)] = v
            return 0
        lax.fori_loop(0, _MSL // _L, _red, 0)

    def _mol(g, _):
        o = g * _L
        ms = sums[0, pl.ds(o, _L)]
        tmpw[0, pl.ds(o, _L)] = sums[1, pl.ds(o, _L)] / ms
        tmpw[1, pl.ds(o, _L)] = sums[2, pl.ds(o, _L)] / ms
        tmpw[2, pl.ds(o, _L)] = sums[3, pl.ds(o, _L)] / ms
        tmpw[3, pl.ds(o, _L)] = sums[4, pl.ds(o, _L)] / sums[5, pl.ds(o, _L)]
        return 0
    lax.fori_loop(0, _MSL // _L, _mol, 0)
    for c in range(4):
        pltpu.sync_copy(tmpw.at[c], spmol.at[c, pl.ds(m0, _MSL)])
    plsc.subcore_barrier()
    pltpu.sync_copy(spmol, molloc)

    def _chunk_b(k, _):
        a0 = base + k * _CH
        pltpu.sync_copy(batch_hbm.at[pl.ds(a0, _CH)], bufb)
        pltpu.sync_copy(z_hbm.at[pl.ds(a0, _CH)], bufz)
        pltpu.sync_copy(px_hbm.at[pl.ds(a0, _CH)], bufpx)
        pltpu.sync_copy(py_hbm.at[pl.ds(a0, _CH)], bufpy)
        pltpu.sync_copy(pz_hbm.at[pl.ds(a0, _CH)], bufpz)
        pltpu.sync_copy(q_hbm.at[pl.ds(a0, _CH)], bufq)

        def _grp(g, _):
            o = g * _L
            b = bufb[pl.ds(o, _L)]
            zv = bufz[pl.ds(o, _L)]
            zf = zv.astype(jnp.float32)
            px = bufpx[pl.ds(o, _L)]
            py = bufpy[pl.ds(o, _L)]
            pz = bufpz[pl.ds(o, _L)]
            qv = bufq[pl.ds(o, _L)]
            cmx = plsc.load_gather(molloc, [rows0, b])
            cmy = plsc.load_gather(molloc, [rows1, b])
            cmz = plsc.load_gather(molloc, [rows2, b])
            mq = plsc.load_gather(molloc, [rows3, b])
            dx = px - cmx
            dy = py - cmy
            dz = pz - cmz
            r2v = (dx * dx + dy * dy + dz * dz) * _A2
            cloud = jnp.abs(qv - mq - zf)
            b_next = b.at[shift_idx].get(mode="promise_in_bounds")
            diff = b != b_next
            _seg_scatter_add(r2loc, b, b_next, diff | lastlane,
                             diff & notlast, cloud * r2v)
            return 0
        lax.fori_loop(0, _NG, _grp, 0)
        return 0
    lax.fori_loop(0, _NCH, _chunk_b, 0)

    pltpu.sync_copy(r2loc, spr2.at[sid])
    plsc.subcore_barrier()
    for t in range(_NS):
        pltpu.sync_copy(spr2.at[t, pl.ds(m0, _MSL)], redbuf.at[t])

    def _redr(g, _):
        o = g * _L
        v = zero16
        for t in range(_NS):
            v = v + redbuf[t, pl.ds(o, _L)]
        res[pl.ds(o, _L)] = v
        return 0
    lax.fori_loop(0, _MSL // _L, _redr, 0)

    @pl.when(sid // (_NS // _NC) == cid)
    def _():
        pltpu.sync_copy(res, out_hbm.at[pl.ds(m0, _MSL)])


_sc_call = pl.kernel(
    _sc_body,
    out_type=jax.ShapeDtypeStruct((_B,), jnp.float32),
    mesh=plsc.VectorSubcoreMesh(core_axis_name="c", subcore_axis_name="s",
                                num_cores=_NC, num_subcores=_NS),
    compiler_params=pltpu.CompilerParams(needs_layout_passes=False),
    scratch_types=[
        pltpu.VMEM((_CH,), jnp.int32),
        pltpu.VMEM((_CH,), jnp.int32),
        pltpu.VMEM((_CH,), jnp.float32),
        pltpu.VMEM((_CH,), jnp.float32),
        pltpu.VMEM((_CH,), jnp.float32),
        pltpu.VMEM((_CH,), jnp.float32),
        pltpu.VMEM((_B,), jnp.float32),
        pltpu.VMEM((_B,), jnp.float32),
        pltpu.VMEM((_B,), jnp.float32),
        pltpu.VMEM((_B,), jnp.float32),
        pltpu.VMEM((_B,), jnp.float32),
        pltpu.VMEM((_B,), jnp.float32),
        pltpu.VMEM((4, _B), jnp.float32),
        pltpu.VMEM((_B,), jnp.float32),
        pltpu.VMEM((_NS, _MSL), jnp.float32),
        pltpu.VMEM((6, _MSL), jnp.float32),
        pltpu.VMEM((4, _MSL), jnp.float32),
        pltpu.VMEM((_MSL,), jnp.float32),
        pltpu.VMEM_SHARED((6, _NS, _B), jnp.float32),
        pltpu.VMEM_SHARED((4, _B), jnp.float32),
        pltpu.VMEM_SHARED((_NS, _B), jnp.float32),
    ],
)


def kernel(x, pos, Z, batch, W0, b0, W1, b1):
    q = _mlp(x, W0, b0.reshape(1, _H), W1, b1.reshape(1, 1))
    pos_t = pos.T
    r2 = _sc_call(batch.astype(jnp.int32), Z[:, 0].astype(jnp.int32),
                  pos_t[0], pos_t[1], pos_t[2], q[:, 0])
    return r2.reshape(_B, 1)
